# baseline (speedup 1.0000x reference)
import functools

import jax
import jax.numpy as jnp
from jax import lax
from jax.experimental import pallas as pl
from jax.experimental.pallas import tpu as pltpu
from jax.experimental.pallas import tpu_sc as plsc

NC = 2
NS = 16
NW = NC * NS
CH = 128
KR = 80
NBUF = 2


def _mesh():
    return plsc.VectorSubcoreMesh(
        core_axis_name="c", subcore_axis_name="s", num_cores=NC, num_subcores=NS
    )


def _build_deg(N_pad):
    spt = N_pad // NS
    BK = 8

    @functools.partial(
        pl.kernel,
        mesh=_mesh(),
        out_type=jax.ShapeDtypeStruct((NC * 2 * N_pad,), jnp.float32),
        scratch_types=[
            pltpu.VMEM((KR, CH), jnp.int32),
            pltpu.VMEM((KR, CH), jnp.int32),
            pltpu.VMEM((CH,), jnp.float32),
            pltpu.VMEM((spt,), jnp.float32),
            pltpu.VMEM_SHARED((N_pad,), jnp.float32),
            pltpu.VMEM_SHARED((N_pad,), jnp.float32),
            pltpu.SemaphoreType.DMA,
            pltpu.SemaphoreType.DMA,
        ],
    )
    def deg_kernel(src_hbm, dst_hbm, out_hbm, sidx_v, didx_v, ones_v, zslice_v,
                   dsrc_sh, ddst_sh, lsem, ssem):
        cid = lax.axis_index("c")
        sid = lax.axis_index("s")
        w = cid * NS + sid

        ld_s = pltpu.async_copy(src_hbm.at[pl.ds(w * KR, KR)], sidx_v, lsem)
        ld_d = pltpu.async_copy(dst_hbm.at[pl.ds(w * KR, KR)], didx_v, lsem)

        @pl.loop(0, CH // 16)
        def _(i):
            ones_v[pl.ds(i * 16, 16)] = jnp.full((16,), 1.0, jnp.float32)

        @pl.loop(0, spt // 16)
        def _(i):
            zslice_v[pl.ds(i * 16, 16)] = jnp.zeros((16,), jnp.float32)

        nb = sid * spt
        pltpu.sync_copy(zslice_v, dsrc_sh.at[pl.ds(nb, spt)])
        pltpu.sync_copy(zslice_v, ddst_sh.at[pl.ds(nb, spt)])
        ld_s.wait()
        ld_d.wait()
        plsc.subcore_barrier()

        @pl.loop(0, KR // BK)
        def _(t):
            descs = []
            for b in range(BK):
                descs.append(pltpu.async_copy(
                    ones_v, dsrc_sh.at[sidx_v.at[t * BK + b]], ssem, add=True))
                descs.append(pltpu.async_copy(
                    ones_v, ddst_sh.at[didx_v.at[t * BK + b]], ssem, add=True))
            for d in descs:
                d.wait()

        plsc.subcore_barrier()
        pltpu.sync_copy(
            dsrc_sh.at[pl.ds(nb, spt)],
            out_hbm.at[pl.ds((cid * 2 + 0) * N_pad + nb, spt)],
        )
        pltpu.sync_copy(
            ddst_sh.at[pl.ds(nb, spt)],
            out_hbm.at[pl.ds((cid * 2 + 1) * N_pad + nb, spt)],
        )

    return deg_kernel


def _build_agg(N_pad, D):
    rpt = N_pad // NS
    IB = 16
    NB_I = KR // IB

    @functools.partial(
        pl.kernel,
        mesh=_mesh(),
        out_type=jax.ShapeDtypeStruct((NC * N_pad, D), jnp.float32),
        scratch_types=[
            pltpu.VMEM((2, IB, CH), jnp.int32),
            pltpu.VMEM((2, IB, CH), jnp.int32),
            pltpu.VMEM((NBUF, CH, D), jnp.float32),
            pltpu.VMEM_SHARED((N_pad, D), jnp.float32),
            pltpu.SemaphoreType.DMA,
            pltpu.SemaphoreType.DMA,
        ]
        + [pltpu.SemaphoreType.DMA] * NBUF
        + [pltpu.SemaphoreType.DMA] * NBUF,
    )
    def agg_kernel(h_hbm, src_hbm, dst_hbm, out_hbm, sidx_v, didx_v, rows_v,
                   acc_sh, *sems):
        lsem = sems[:2]
        gsem = sems[2 : 2 + NBUF]
        ssem = sems[2 + NBUF :]
        cid = lax.axis_index("c")
        sid = lax.axis_index("s")
        w = cid * NS + sid
        rowbase = w * KR

        pltpu.async_copy(src_hbm.at[pl.ds(rowbase, IB)], sidx_v.at[0], lsem[0])
        pltpu.async_copy(dst_hbm.at[pl.ds(rowbase, IB)], didx_v.at[0], lsem[0])

        @pl.loop(0, CH)
        def _(r):
            for k in range(D // 16):
                rows_v[0, r, pl.ds(k * 16, 16)] = jnp.zeros((16,), jnp.float32)

        rb = sid * rpt

        @pl.loop(0, rpt // CH)
        def _(t):
            pltpu.sync_copy(rows_v.at[0], acc_sh.at[pl.ds(rb + t * CH, CH)])

        plsc.subcore_barrier()

        for tb in range(NB_I):
            slot = tb % 2
            pltpu.make_async_copy(
                src_hbm.at[pl.ds(0, IB)], sidx_v.at[slot], lsem[slot]
            ).wait()
            pltpu.make_async_copy(
                dst_hbm.at[pl.ds(0, IB)], didx_v.at[slot], lsem[slot]
            ).wait()
            if tb + 1 < NB_I:
                nb_base = rowbase + (tb + 1) * IB
                pltpu.async_copy(
                    src_hbm.at[pl.ds(nb_base, IB)], sidx_v.at[1 - slot], lsem[1 - slot])
                pltpu.async_copy(
                    dst_hbm.at[pl.ds(nb_base, IB)], didx_v.at[1 - slot], lsem[1 - slot])

            def fire_gather(b, r):
                pltpu.async_copy(
                    h_hbm.at[sidx_v.at[slot, r, pl.ds(0, CH // 2)]],
                    rows_v.at[b, pl.ds(0, CH // 2)], gsem[b])
                pltpu.async_copy(
                    h_hbm.at[sidx_v.at[slot, r, pl.ds(CH // 2, CH // 2)]],
                    rows_v.at[b, pl.ds(CH // 2, CH // 2)], gsem[b])

            def wait_gather(b):
                pltpu.make_async_copy(
                    h_hbm.at[pl.ds(0, CH)], rows_v.at[b], gsem[b]).wait()

            if tb == 0:
                for b in range(NBUF):
                    fire_gather(b, b)
                for b in range(NBUF):
                    wait_gather(b)
                    pltpu.async_copy(
                        rows_v.at[b], acc_sh.at[didx_v.at[slot, b]], ssem[b], add=True)
                lo = 1
            else:
                lo = 0

            @pl.loop(lo, IB // NBUF)
            def _(u):
                for b in range(NBUF):
                    pltpu.make_async_copy(
                        rows_v.at[b], acc_sh.at[pl.ds(0, CH)], ssem[b]).wait()
                    fire_gather(b, u * NBUF + b)
                for b in range(NBUF):
                    wait_gather(b)
                    pltpu.async_copy(
                        rows_v.at[b], acc_sh.at[didx_v.at[slot, u * NBUF + b]],
                        ssem[b], add=True)

        for b in range(NBUF):
            pltpu.make_async_copy(rows_v.at[b], acc_sh.at[pl.ds(0, CH)], ssem[b]).wait()

        plsc.subcore_barrier()
        pltpu.sync_copy(
            acc_sh.at[pl.ds(rb, rpt)],
            out_hbm.at[pl.ds(cid * N_pad + rb, rpt)],
        )

    return agg_kernel


def _h_body(degt_ref, f_ref, h_ref):
    dsrc = degt_ref[:, 0] + degt_ref[:, 2]
    norm = lax.rsqrt(jnp.maximum(dsrc, 1.0))
    h_ref[...] = f_ref[...] * norm[:, None]


def _final_body(aggp_ref, degt_ref, f_ref, wc_ref, bc_ref, wa_ref, ba_ref, o_ref):
    agg = aggp_ref[0] + aggp_ref[1]
    din = degt_ref[:, 1] + degt_ref[:, 3]
    norm = lax.rsqrt(jnp.maximum(din, 1.0))
    agg = agg * norm[:, None]
    conv = jnp.dot(agg, wc_ref[...], preferred_element_type=jnp.float32)
    conv = conv + bc_ref[...][None, :]
    D = f_ref.shape[1]
    out = jnp.dot(conv, wa_ref[0:D, :], preferred_element_type=jnp.float32)
    out = out + jnp.dot(f_ref[...], wa_ref[D : 2 * D, :], preferred_element_type=jnp.float32)
    o_ref[...] = out + ba_ref[...][None, :]


def kernel(features, edge_index, W_conv, b_conv, W_aggr, b_aggr):
    N, D = features.shape
    E = edge_index.shape[1]
    N_pad = ((N + 2 * NW * 16 - 1) // (2 * NW * 16)) * (2 * NW * 16)
    E_pad = NW * KR * CH
    BLK = 1000

    src = edge_index[0]
    dst = edge_index[1]
    trash = N + (jnp.arange(E_pad - E, dtype=jnp.int32) & 127)
    src_p = jnp.concatenate([src, trash]).reshape(NW * KR, CH)
    dst_p = jnp.concatenate([dst, trash]).reshape(NW * KR, CH)

    degt = _build_deg(N_pad)(src_p, dst_p).reshape(NC * 2, N_pad).T

    h = pl.pallas_call(
        _h_body,
        grid=(N // BLK,),
        in_specs=[
            pl.BlockSpec((BLK, NC * 2), lambda i: (i, 0)),
            pl.BlockSpec((BLK, D), lambda i: (i, 0)),
        ],
        out_specs=pl.BlockSpec((BLK, D), lambda i: (i, 0)),
        out_shape=jax.ShapeDtypeStruct((N, D), jnp.float32),
    )(degt, features)

    aggp = _build_agg(N_pad, D)(h, src_p, dst_p).reshape(NC, N_pad, D)

    out = pl.pallas_call(
        _final_body,
        grid=(N // BLK,),
        in_specs=[
            pl.BlockSpec((NC, BLK, D), lambda i: (0, i, 0)),
            pl.BlockSpec((BLK, NC * 2), lambda i: (i, 0)),
            pl.BlockSpec((BLK, D), lambda i: (i, 0)),
            pl.BlockSpec((D, D), lambda i: (0, 0)),
            pl.BlockSpec((D,), lambda i: (0,)),
            pl.BlockSpec((2 * D, D), lambda i: (0, 0)),
            pl.BlockSpec((D,), lambda i: (0,)),
        ],
        out_specs=pl.BlockSpec((BLK, D), lambda i: (i, 0)),
        out_shape=jax.ShapeDtypeStruct((N, D), jnp.float32),
    )(aggp, degt, features, W_conv, b_conv, W_aggr, b_aggr)

    return out


# speedup vs baseline: 1.0478x; 1.0478x over previous
import functools

import jax
import jax.numpy as jnp
from jax import lax
from jax.experimental import pallas as pl
from jax.experimental.pallas import tpu as pltpu
from jax.experimental.pallas import tpu_sc as plsc

NC = 2
NS = 16
NW = NC * NS
CH = 128
NBUF = 2


def _mesh():
    return plsc.VectorSubcoreMesh(
        core_axis_name="c", subcore_axis_name="s", num_cores=NC, num_subcores=NS
    )


def _build_deg(N_pad, E):
    spt = N_pad // NS
    n_chunks = E // CH
    base_per, extra = divmod(n_chunks, NW)
    BK = 6
    nbatch = base_per // BK

    @functools.partial(
        pl.kernel,
        mesh=_mesh(),
        out_type=jax.ShapeDtypeStruct((NC * 2 * N_pad,), jnp.float32),
        scratch_types=[
            pltpu.VMEM((2, BK, 2, CH), jnp.int32),
            pltpu.VMEM((CH,), jnp.float32),
            pltpu.VMEM((spt,), jnp.float32),
            pltpu.VMEM_SHARED((N_pad,), jnp.float32),
            pltpu.VMEM_SHARED((N_pad,), jnp.float32),
            pltpu.SemaphoreType.DMA,
            pltpu.SemaphoreType.DMA,
            pltpu.SemaphoreType.DMA,
        ],
    )
    def deg_kernel(edge_hbm, out_hbm, idx_v, ones_v, zslice_v,
                   dsrc_sh, ddst_sh, lsem0, lsem1, ssem):
        lsem = (lsem0, lsem1)
        cid = lax.axis_index("c")
        sid = lax.axis_index("s")
        w = cid * NS + sid
        cb = w * base_per + jnp.minimum(w, extra)

        def load_batch(slot, t):
            for r in range(BK):
                pltpu.async_copy(
                    edge_hbm.at[:, pl.ds((cb + t * BK + r) * CH, CH)],
                    idx_v.at[slot, r], lsem[slot])

        def wait_batch(slot):
            for r in range(BK):
                pltpu.make_async_copy(
                    edge_hbm.at[:, pl.ds(0, CH)], idx_v.at[slot, r], lsem[slot]
                ).wait()

        load_batch(0, 0)

        @pl.loop(0, CH // 16)
        def _(i):
            ones_v[pl.ds(i * 16, 16)] = jnp.full((16,), 1.0, jnp.float32)

        @pl.loop(0, spt // 16)
        def _(i):
            zslice_v[pl.ds(i * 16, 16)] = jnp.zeros((16,), jnp.float32)

        nb = sid * spt
        pltpu.sync_copy(zslice_v, dsrc_sh.at[pl.ds(nb, spt)])
        pltpu.sync_copy(zslice_v, ddst_sh.at[pl.ds(nb, spt)])
        plsc.subcore_barrier()

        for t in range(nbatch):
            slot = t % 2
            wait_batch(slot)
            if t + 1 < nbatch:
                load_batch(1 - slot, t + 1)
            descs = []
            for r in range(BK):
                descs.append(pltpu.async_copy(
                    ones_v, dsrc_sh.at[idx_v.at[slot, r, 0]], ssem, add=True))
                descs.append(pltpu.async_copy(
                    ones_v, ddst_sh.at[idx_v.at[slot, r, 1]], ssem, add=True))
            for d in descs:
                d.wait()

        @pl.when(w < extra)
        def _():
            pltpu.sync_copy(
                edge_hbm.at[:, pl.ds((cb + base_per) * CH, CH)], idx_v.at[0, 0])
            d1 = pltpu.async_copy(
                ones_v, dsrc_sh.at[idx_v.at[0, 0, 0]], ssem, add=True)
            d2 = pltpu.async_copy(
                ones_v, ddst_sh.at[idx_v.at[0, 0, 1]], ssem, add=True)
            d1.wait()
            d2.wait()

        plsc.subcore_barrier()
        pltpu.sync_copy(
            dsrc_sh.at[pl.ds(nb, spt)],
            out_hbm.at[pl.ds((cid * 2 + 0) * N_pad + nb, spt)],
        )
        pltpu.sync_copy(
            ddst_sh.at[pl.ds(nb, spt)],
            out_hbm.at[pl.ds((cid * 2 + 1) * N_pad + nb, spt)],
        )

    return deg_kernel


def _build_agg(N_pad, D, E):
    rpt = N_pad // NS
    IB = 16
    n_chunks = E // CH
    base_per, extra = divmod(n_chunks, NW)
    blocks = [IB] * (base_per // IB)
    if base_per % IB:
        blocks.append(base_per % IB)
    starts = [sum(blocks[:i]) for i in range(len(blocks))]

    @functools.partial(
        pl.kernel,
        mesh=_mesh(),
        out_type=jax.ShapeDtypeStruct((NC * N_pad, D), jnp.float32),
        scratch_types=[
            pltpu.VMEM((2, IB, 2, CH), jnp.int32),
            pltpu.VMEM((NBUF, CH, D), jnp.float32),
            pltpu.VMEM_SHARED((N_pad, D), jnp.float32),
            pltpu.SemaphoreType.DMA,
            pltpu.SemaphoreType.DMA,
        ]
        + [pltpu.SemaphoreType.DMA] * NBUF
        + [pltpu.SemaphoreType.DMA] * NBUF,
    )
    def agg_kernel(h_hbm, edge_hbm, out_hbm, idx_v, rows_v, acc_sh, *sems):
        lsem = sems[:2]
        gsem = sems[2 : 2 + NBUF]
        ssem = sems[2 + NBUF :]
        cid = lax.axis_index("c")
        sid = lax.axis_index("s")
        w = cid * NS + sid
        cb = w * base_per + jnp.minimum(w, extra)

        def load_block(slot, tb):
            for r in range(blocks[tb]):
                pltpu.async_copy(
                    edge_hbm.at[:, pl.ds((cb + starts[tb] + r) * CH, CH)],
                    idx_v.at[slot, r], lsem[slot])

        def wait_block(slot, tb):
            for r in range(blocks[tb]):
                pltpu.make_async_copy(
                    edge_hbm.at[:, pl.ds(0, CH)], idx_v.at[slot, r], lsem[slot]
                ).wait()

        load_block(0, 0)

        @pl.loop(0, CH)
        def _(r):
            for k in range(D // 16):
                rows_v[0, r, pl.ds(k * 16, 16)] = jnp.zeros((16,), jnp.float32)

        rb = sid * rpt

        @pl.loop(0, rpt // CH)
        def _(t):
            pltpu.sync_copy(rows_v.at[0], acc_sh.at[pl.ds(rb + t * CH, CH)])

        plsc.subcore_barrier()

        def fire_gather(slot, r, b):
            pltpu.async_copy(
                h_hbm.at[idx_v.at[slot, r, 0]], rows_v.at[b], gsem[b])

        def wait_gather(b):
            pltpu.make_async_copy(
                h_hbm.at[pl.ds(0, CH)], rows_v.at[b], gsem[b]).wait()

        def fire_scatter(slot, r, b):
            pltpu.async_copy(
                rows_v.at[b], acc_sh.at[idx_v.at[slot, r, 1]], ssem[b], add=True)

        def wait_scatter(b):
            pltpu.make_async_copy(
                rows_v.at[b], acc_sh.at[pl.ds(0, CH)], ssem[b]).wait()

        for tb in range(len(blocks)):
            slot = tb % 2
            wait_block(slot, tb)
            if tb + 1 < len(blocks):
                load_block(1 - slot, tb + 1)

            if tb == 0:
                for b in range(NBUF):
                    fire_gather(slot, b, b)
                for b in range(NBUF):
                    wait_gather(b)
                    fire_scatter(slot, b, b)
                lo = 1
            else:
                lo = 0

            @pl.loop(lo, blocks[tb] // NBUF)
            def _(u):
                for b in range(NBUF):
                    wait_scatter(b)
                    fire_gather(slot, u * NBUF + b, b)
                for b in range(NBUF):
                    wait_gather(b)
                    fire_scatter(slot, u * NBUF + b, b)

        @pl.when(w < extra)
        def _():
            pltpu.sync_copy(
                edge_hbm.at[:, pl.ds((cb + base_per) * CH, CH)], idx_v.at[0, 0])
            wait_scatter(0)
            fire_gather(0, 0, 0)
            wait_gather(0)
            fire_scatter(0, 0, 0)

        for b in range(NBUF):
            wait_scatter(b)

        plsc.subcore_barrier()
        pltpu.sync_copy(
            acc_sh.at[pl.ds(rb, rpt)],
            out_hbm.at[pl.ds(cid * N_pad + rb, rpt)],
        )

    return agg_kernel


def _h_body(degt_ref, f_ref, h_ref):
    dsrc = degt_ref[:, 0] + degt_ref[:, 2]
    norm = lax.rsqrt(jnp.maximum(dsrc, 1.0))
    h_ref[...] = f_ref[...] * norm[:, None]


def _final_body(aggp_ref, degt_ref, f_ref, wc_ref, bc_ref, wa_ref, ba_ref, o_ref):
    agg = aggp_ref[0] + aggp_ref[1]
    din = degt_ref[:, 1] + degt_ref[:, 3]
    norm = lax.rsqrt(jnp.maximum(din, 1.0))
    agg = agg * norm[:, None]
    conv = jnp.dot(agg, wc_ref[...], preferred_element_type=jnp.float32)
    conv = conv + bc_ref[...][None, :]
    D = f_ref.shape[1]
    out = jnp.dot(conv, wa_ref[0:D, :], preferred_element_type=jnp.float32)
    out = out + jnp.dot(f_ref[...], wa_ref[D : 2 * D, :], preferred_element_type=jnp.float32)
    o_ref[...] = out + ba_ref[...][None, :]


def kernel(features, edge_index, W_conv, b_conv, W_aggr, b_aggr):
    N, D = features.shape
    E = edge_index.shape[1]
    N_pad = ((N + 2 * NW * 16 - 1) // (2 * NW * 16)) * (2 * NW * 16)
    BLK = 1000

    degt = _build_deg(N_pad, E)(edge_index).reshape(NC * 2, N_pad).T

    h = pl.pallas_call(
        _h_body,
        grid=(N // BLK,),
        in_specs=[
            pl.BlockSpec((BLK, NC * 2), lambda i: (i, 0)),
            pl.BlockSpec((BLK, D), lambda i: (i, 0)),
        ],
        out_specs=pl.BlockSpec((BLK, D), lambda i: (i, 0)),
        out_shape=jax.ShapeDtypeStruct((N, D), jnp.float32),
    )(degt, features)

    aggp = _build_agg(N_pad, D, E)(h, edge_index).reshape(NC, N_pad, D)

    out = pl.pallas_call(
        _final_body,
        grid=(N // BLK,),
        in_specs=[
            pl.BlockSpec((NC, BLK, D), lambda i: (0, i, 0)),
            pl.BlockSpec((BLK, NC * 2), lambda i: (i, 0)),
            pl.BlockSpec((BLK, D), lambda i: (i, 0)),
            pl.BlockSpec((D, D), lambda i: (0, 0)),
            pl.BlockSpec((D,), lambda i: (0,)),
            pl.BlockSpec((2 * D, D), lambda i: (0, 0)),
            pl.BlockSpec((D,), lambda i: (0,)),
        ],
        out_specs=pl.BlockSpec((BLK, D), lambda i: (i, 0)),
        out_shape=jax.ShapeDtypeStruct((N, D), jnp.float32),
    )(aggp, degt, features, W_conv, b_conv, W_aggr, b_aggr)

    return out


# speedup vs baseline: 1.0500x; 1.0021x over previous
import functools

import jax
import jax.numpy as jnp
from jax import lax
from jax.experimental import pallas as pl
from jax.experimental.pallas import tpu as pltpu
from jax.experimental.pallas import tpu_sc as plsc

NC = 2
NS = 16
NW = NC * NS
CH = 128
NBUF = 2


def _mesh():
    return plsc.VectorSubcoreMesh(
        core_axis_name="c", subcore_axis_name="s", num_cores=NC, num_subcores=NS
    )


def _build_deg(N_pad, E):
    spt = N_pad // NS
    n_chunks = E // CH
    base_per, extra = divmod(n_chunks, NW)
    BK = 6
    nbatch = base_per // BK

    @functools.partial(
        pl.kernel,
        mesh=_mesh(),
        out_type=jax.ShapeDtypeStruct((NC * 2 * N_pad,), jnp.float32),
        scratch_types=[
            pltpu.VMEM((2, BK, 2, CH), jnp.int32),
            pltpu.VMEM((CH,), jnp.float32),
            pltpu.VMEM((spt,), jnp.float32),
            pltpu.VMEM_SHARED((N_pad,), jnp.float32),
            pltpu.VMEM_SHARED((N_pad,), jnp.float32),
            pltpu.SemaphoreType.DMA,
            pltpu.SemaphoreType.DMA,
            pltpu.SemaphoreType.DMA,
        ],
    )
    def deg_kernel(edge_hbm, out_hbm, idx_v, ones_v, zslice_v,
                   dsrc_sh, ddst_sh, lsem0, lsem1, ssem):
        lsem = (lsem0, lsem1)
        cid = lax.axis_index("c")
        sid = lax.axis_index("s")
        w = cid * NS + sid
        cb = w * base_per + jnp.minimum(w, extra)

        def load_batch(slot, t):
            for r in range(BK):
                pltpu.async_copy(
                    edge_hbm.at[:, pl.ds((cb + t * BK + r) * CH, CH)],
                    idx_v.at[slot, r], lsem[slot])

        def wait_batch(slot):
            for r in range(BK):
                pltpu.make_async_copy(
                    edge_hbm.at[:, pl.ds(0, CH)], idx_v.at[slot, r], lsem[slot]
                ).wait()

        load_batch(0, 0)

        @pl.loop(0, CH // 16)
        def _(i):
            ones_v[pl.ds(i * 16, 16)] = jnp.full((16,), 1.0, jnp.float32)

        @pl.loop(0, spt // 16)
        def _(i):
            zslice_v[pl.ds(i * 16, 16)] = jnp.zeros((16,), jnp.float32)

        nb = sid * spt
        pltpu.sync_copy(zslice_v, dsrc_sh.at[pl.ds(nb, spt)])
        pltpu.sync_copy(zslice_v, ddst_sh.at[pl.ds(nb, spt)])
        plsc.subcore_barrier()

        for t in range(nbatch):
            slot = t % 2
            wait_batch(slot)
            if t + 1 < nbatch:
                load_batch(1 - slot, t + 1)
            descs = []
            for r in range(BK):
                descs.append(pltpu.async_copy(
                    ones_v, dsrc_sh.at[idx_v.at[slot, r, 0]], ssem, add=True))
                descs.append(pltpu.async_copy(
                    ones_v, ddst_sh.at[idx_v.at[slot, r, 1]], ssem, add=True))
            for d in descs:
                d.wait()

        @pl.when(w < extra)
        def _():
            pltpu.sync_copy(
                edge_hbm.at[:, pl.ds((cb + base_per) * CH, CH)], idx_v.at[0, 0])
            d1 = pltpu.async_copy(
                ones_v, dsrc_sh.at[idx_v.at[0, 0, 0]], ssem, add=True)
            d2 = pltpu.async_copy(
                ones_v, ddst_sh.at[idx_v.at[0, 0, 1]], ssem, add=True)
            d1.wait()
            d2.wait()

        plsc.subcore_barrier()
        pltpu.sync_copy(
            dsrc_sh.at[pl.ds(nb, spt)],
            out_hbm.at[pl.ds((cid * 2 + 0) * N_pad + nb, spt)],
        )
        pltpu.sync_copy(
            ddst_sh.at[pl.ds(nb, spt)],
            out_hbm.at[pl.ds((cid * 2 + 1) * N_pad + nb, spt)],
        )

    return deg_kernel


def _build_agg(N_pad, D, E):
    rpt = N_pad // NS
    IB = 16
    n_chunks = E // CH
    base_per, extra = divmod(n_chunks, NW)
    blocks = [IB] * (base_per // IB)
    if base_per % IB:
        blocks.append(base_per % IB)
    starts = [sum(blocks[:i]) for i in range(len(blocks))]

    @functools.partial(
        pl.kernel,
        mesh=_mesh(),
        out_type=jax.ShapeDtypeStruct((NC * N_pad, D), jnp.float32),
        scratch_types=[
            pltpu.VMEM((2, IB, 2, CH), jnp.int32),
            pltpu.VMEM((NBUF, CH, D), jnp.float32),
            pltpu.VMEM_SHARED((N_pad, D), jnp.float32),
            pltpu.SemaphoreType.DMA,
            pltpu.SemaphoreType.DMA,
        ]
        + [pltpu.SemaphoreType.DMA] * NBUF
        + [pltpu.SemaphoreType.DMA] * NBUF,
    )
    def agg_kernel(h_hbm, edge_hbm, out_hbm, idx_v, rows_v, acc_sh, *sems):
        lsem = sems[:2]
        gsem = sems[2 : 2 + NBUF]
        ssem = sems[2 + NBUF :]
        cid = lax.axis_index("c")
        sid = lax.axis_index("s")
        w = cid * NS + sid
        cb = w * base_per + jnp.minimum(w, extra)

        def load_block(slot, tb):
            for r in range(blocks[tb]):
                pltpu.async_copy(
                    edge_hbm.at[:, pl.ds((cb + starts[tb] + r) * CH, CH)],
                    idx_v.at[slot, r], lsem[slot])

        def wait_block(slot, tb):
            for r in range(blocks[tb]):
                pltpu.make_async_copy(
                    edge_hbm.at[:, pl.ds(0, CH)], idx_v.at[slot, r], lsem[slot]
                ).wait()

        load_block(0, 0)

        @pl.loop(0, CH)
        def _(r):
            for k in range(D // 16):
                rows_v[0, r, pl.ds(k * 16, 16)] = jnp.zeros((16,), jnp.float32)

        rb = sid * rpt

        zds = [
            pltpu.async_copy(
                rows_v.at[0], acc_sh.at[pl.ds(rb + t * CH, CH)], gsem[0])
            for t in range(rpt // CH)
        ]
        for d in zds:
            d.wait()

        plsc.subcore_barrier()

        def fire_gather(slot, r, b):
            pltpu.async_copy(
                h_hbm.at[idx_v.at[slot, r, 0]], rows_v.at[b], gsem[b])

        def wait_gather(b):
            pltpu.make_async_copy(
                h_hbm.at[pl.ds(0, CH)], rows_v.at[b], gsem[b]).wait()

        def fire_scatter(slot, r, b):
            pltpu.async_copy(
                rows_v.at[b], acc_sh.at[idx_v.at[slot, r, 1]], ssem[b], add=True)

        def wait_scatter(b):
            pltpu.make_async_copy(
                rows_v.at[b], acc_sh.at[pl.ds(0, CH)], ssem[b]).wait()

        for tb in range(len(blocks)):
            slot = tb % 2
            wait_block(slot, tb)
            if tb + 1 < len(blocks):
                load_block(1 - slot, tb + 1)

            if tb == 0:
                for b in range(NBUF):
                    fire_gather(slot, b, b)
                for b in range(NBUF):
                    wait_gather(b)
                    fire_scatter(slot, b, b)
                lo = 1
            else:
                lo = 0

            @pl.loop(lo, blocks[tb] // NBUF)
            def _(u):
                for b in range(NBUF):
                    wait_scatter(b)
                    fire_gather(slot, u * NBUF + b, b)
                for b in range(NBUF):
                    wait_gather(b)
                    fire_scatter(slot, u * NBUF + b, b)

        @pl.when(w < extra)
        def _():
            pltpu.sync_copy(
                edge_hbm.at[:, pl.ds((cb + base_per) * CH, CH)], idx_v.at[0, 0])
            wait_scatter(0)
            fire_gather(0, 0, 0)
            wait_gather(0)
            fire_scatter(0, 0, 0)

        for b in range(NBUF):
            wait_scatter(b)

        plsc.subcore_barrier()
        pltpu.sync_copy(
            acc_sh.at[pl.ds(rb, rpt)],
            out_hbm.at[pl.ds(cid * N_pad + rb, rpt)],
        )

    return agg_kernel


def _h_body(degt_ref, f_ref, h_ref):
    dsrc = degt_ref[:, 0] + degt_ref[:, 2]
    norm = lax.rsqrt(jnp.maximum(dsrc, 1.0))
    h_ref[...] = f_ref[...] * norm[:, None]


def _final_body(aggp_ref, degt_ref, f_ref, wc_ref, bc_ref, wa_ref, ba_ref, o_ref):
    agg = aggp_ref[0] + aggp_ref[1]
    din = degt_ref[:, 1] + degt_ref[:, 3]
    norm = lax.rsqrt(jnp.maximum(din, 1.0))
    agg = agg * norm[:, None]
    conv = jnp.dot(agg, wc_ref[...], preferred_element_type=jnp.float32)
    conv = conv + bc_ref[...][None, :]
    D = f_ref.shape[1]
    out = jnp.dot(conv, wa_ref[0:D, :], preferred_element_type=jnp.float32)
    out = out + jnp.dot(f_ref[...], wa_ref[D : 2 * D, :], preferred_element_type=jnp.float32)
    o_ref[...] = out + ba_ref[...][None, :]


def kernel(features, edge_index, W_conv, b_conv, W_aggr, b_aggr):
    N, D = features.shape
    E = edge_index.shape[1]
    N_pad = ((N + 2 * NW * 16 - 1) // (2 * NW * 16)) * (2 * NW * 16)
    BLK = 1000

    degt = _build_deg(N_pad, E)(edge_index).reshape(NC * 2, N_pad).T

    h = pl.pallas_call(
        _h_body,
        grid=(N // BLK,),
        in_specs=[
            pl.BlockSpec((BLK, NC * 2), lambda i: (i, 0)),
            pl.BlockSpec((BLK, D), lambda i: (i, 0)),
        ],
        out_specs=pl.BlockSpec((BLK, D), lambda i: (i, 0)),
        out_shape=jax.ShapeDtypeStruct((N, D), jnp.float32),
    )(degt, features)

    aggp = _build_agg(N_pad, D, E)(h, edge_index).reshape(NC, N_pad, D)

    out = pl.pallas_call(
        _final_body,
        grid=(N // BLK,),
        in_specs=[
            pl.BlockSpec((NC, BLK, D), lambda i: (0, i, 0)),
            pl.BlockSpec((BLK, NC * 2), lambda i: (i, 0)),
            pl.BlockSpec((BLK, D), lambda i: (i, 0)),
            pl.BlockSpec((D, D), lambda i: (0, 0)),
            pl.BlockSpec((D,), lambda i: (0,)),
            pl.BlockSpec((2 * D, D), lambda i: (0, 0)),
            pl.BlockSpec((D,), lambda i: (0,)),
        ],
        out_specs=pl.BlockSpec((BLK, D), lambda i: (i, 0)),
        out_shape=jax.ShapeDtypeStruct((N, D), jnp.float32),
    )(aggp, degt, features, W_conv, b_conv, W_aggr, b_aggr)

    return out
